# baseline (speedup 1.0000x reference)
import functools

import jax
import jax.numpy as jnp
from jax import lax
from jax.experimental import pallas as pl
from jax.experimental.pallas import tpu as pltpu
from jax.experimental.pallas import tpu_sc as plsc

NC = 2
NS = 16
NW = NC * NS


@functools.lru_cache(maxsize=None)
def _make_lookup(bsz, hist, v4):
    tblk = bsz // 128
    tl = tblk // NW
    nu = hist * tl
    cols = tl * 128
    mesh = plsc.VectorSubcoreMesh(core_axis_name="c", subcore_axis_name="s")

    @functools.partial(
        pl.kernel,
        mesh=mesh,
        out_type=jax.ShapeDtypeStruct((hist, 32, bsz), jnp.float32),
        scratch_types=(
            [pltpu.VMEM((hist * cols,), jnp.int32),
             pltpu.VMEM((128,), jnp.int32)]
            + [pltpu.VMEM((128,), jnp.int32) for _ in range(2)]
            + [pltpu.VMEM((128,), jnp.int32) for _ in range(2)]
            + [pltpu.VMEM((128, 128), jnp.float32) for _ in range(2)]
            + [pltpu.VMEM((32, 128), jnp.float32) for _ in range(2)]
            + [pltpu.SemaphoreType.DMA for _ in range(5)]
        ),
        compiler_params=pltpu.CompilerParams(needs_layout_passes=False),
    )
    def k(table4_hbm, idx_hbm, out_hbm, idx_all, riota,
          sidx0, sidx1, colb0, colb1, rows0, rows1, outt0, outt1,
          isem, gsem0, gsem1, ssem0, ssem1):
        w = lax.axis_index("s") * NC + lax.axis_index("c")
        col0 = w * cols
        sidx = (sidx0, sidx1)
        colb = (colb0, colb1)
        rows = (rows0, rows1)
        outt = (outt0, outt1)
        gsem = (gsem0, gsem1)
        ssem = (ssem0, ssem1)

        for hh in range(hist):
            pltpu.async_copy(
                idx_hbm.at[pl.ds(hh * bsz + col0, cols)],
                idx_all.at[pl.ds(hh * cols, cols)], isem)
        iot = lax.iota(jnp.int32, 16)
        for jg in range(8):
            riota[pl.ds(jg * 16, 16)] = iot + jg * 16
        for hh in range(hist):
            pltpu.make_async_copy(
                idx_hbm.at[pl.ds(0, cols)], idx_all.at[pl.ds(0, cols)],
                isem).wait()

        def prep_fire(u, s):
            h = u // tl
            tloc = u - h * tl
            base = h * cols + tloc * 128
            for jg in range(8):
                iv = idx_all[pl.ds(base + jg * 16, 16)]
                sidx[s][pl.ds(jg * 16, 16)] = lax.shift_right_logical(iv, 2)
                colb[s][pl.ds(jg * 16, 16)] = lax.shift_left(iv & 3, 5)
            pltpu.async_copy(table4_hbm.at[sidx[s]], rows[s], gsem[s])

        def proc(u, s, p):
            pltpu.make_async_copy(
                table4_hbm.at[sidx[s]], rows[s], gsem[s]).wait()

            @pl.when(p >= 1)
            def _():
                pltpu.make_async_copy(
                    outt[s], out_hbm.at[0, pl.ds(0, 32), pl.ds(0, 128)],
                    ssem[s]).wait()
            for jg in range(8):
                riv = riota[pl.ds(jg * 16, 16)]
                cb = colb[s][pl.ds(jg * 16, 16)]
                for c in range(32):
                    g = plsc.load_gather(rows[s], [riv, cb + c])
                    outt[s][c, pl.ds(jg * 16, 16)] = g
            h = u // tl
            tloc = u - h * tl
            bcol = col0 + tloc * 128
            pltpu.async_copy(
                outt[s], out_hbm.at[h, pl.ds(0, 32), pl.ds(bcol, 128)],
                ssem[s])

        prep_fire(0, 0)

        def body(p, carry):
            u0 = 2 * p
            prep_fire(u0 + 1, 1)
            proc(u0, 0, p)

            @pl.when(u0 + 2 < nu)
            def _():
                prep_fire(u0 + 2, 0)
            proc(u0 + 1, 1, p)
            return carry

        lax.fori_loop(0, nu // 2, body, 0)
        for s in range(2):
            pltpu.make_async_copy(
                outt[s], out_hbm.at[0, pl.ds(0, 32), pl.ds(0, 128)],
                ssem[s]).wait()

    return k


def kernel(item_inputs, table):
    b, h = item_inputs.shape
    v, d = table.shape
    table4 = table.reshape(v // 4, d * 4)
    idx1 = item_inputs.T.reshape(b * h).astype(jnp.int32)
    out3 = _make_lookup(b, h, v // 4)(table4, idx1)
    return out3.transpose(2, 0, 1)


# speedup vs baseline: 1.6044x; 1.6044x over previous
import functools

import jax
import jax.numpy as jnp
from jax import lax
from jax.experimental import pallas as pl
from jax.experimental.pallas import tpu as pltpu
from jax.experimental.pallas import tpu_sc as plsc

NC = 2
NS = 16
NW = NC * NS
PITCH = 129


@functools.lru_cache(maxsize=None)
def _make_lookup(bsz, hist):
    tblk = bsz // 128
    tl = tblk // NW
    nu = hist * tl
    cols = tl * 128
    mesh = plsc.VectorSubcoreMesh(core_axis_name="c", subcore_axis_name="s")

    @functools.partial(
        pl.kernel,
        mesh=mesh,
        out_type=jax.ShapeDtypeStruct((hist, 4, tblk, 8, 128), jnp.float32),
        scratch_types=(
            [pltpu.VMEM((hist * cols,), jnp.int32)]
            + [pltpu.VMEM((128, 32), jnp.float32) for _ in range(2)]
            + [pltpu.VMEM((32, PITCH), jnp.float32) for _ in range(2)]
            + [pltpu.SemaphoreType.DMA for _ in range(5)]
        ),
        compiler_params=pltpu.CompilerParams(
            use_tc_tiling_on_sc=False, needs_layout_passes=False),
    )
    def k(table_hbm, idx_hbm, out_hbm, idx_all,
          rows0, rows1, outt0, outt1, isem, gsem0, gsem1, ssem0, ssem1):
        w = lax.axis_index("s") * NC + lax.axis_index("c")
        col0 = w * cols
        rows = (rows0, rows1)
        outt = (outt0, outt1)
        gsem = (gsem0, gsem1)
        ssem = (ssem0, ssem1)

        for hh in range(hist):
            pltpu.async_copy(
                idx_hbm.at[pl.ds(hh * bsz + col0, cols)],
                idx_all.at[pl.ds(hh * cols, cols)], isem)
        iot = lax.iota(jnp.int32, 16)
        for hh in range(hist):
            pltpu.make_async_copy(
                idx_hbm.at[pl.ds(0, cols)], idx_all.at[pl.ds(0, cols)],
                isem).wait()

        def fire_gather(u, s):
            base = (u // tl) * cols + (u - (u // tl) * tl) * 128
            pltpu.async_copy(
                table_hbm.at[idx_all.at[pl.ds(base, 128)]], rows[s], gsem[s])

        def drain_write(s):
            for r in range(4):
                pltpu.make_async_copy(
                    outt[s].at[pl.ds(0, 8), pl.ds(0, 128)],
                    out_hbm.at[0, 0, 0], ssem[s]).wait()

        def proc(u, s, p):
            pltpu.make_async_copy(
                table_hbm.at[idx_all.at[pl.ds(0, 128)]], rows[s],
                gsem[s]).wait()

            @pl.when(p >= 1)
            def _():
                drain_write(s)
            for j in range(128):
                bi = iot * 0 + j
                for c0 in (0, 16):
                    v = rows[s][j, pl.ds(c0, 16)]
                    plsc.store_scatter(outt[s], [iot + c0, bi], v)
            h = u // tl
            tloc = u - h * tl
            t = (col0 // 128) + tloc
            for r in range(4):
                pltpu.async_copy(
                    outt[s].at[pl.ds(r * 8, 8), pl.ds(0, 128)],
                    out_hbm.at[h, r, t], ssem[s])

        fire_gather(0, 0)

        def body(p, carry):
            u0 = 2 * p
            fire_gather(u0 + 1, 1)
            proc(u0, 0, p)

            @pl.when(u0 + 2 < nu)
            def _():
                fire_gather(u0 + 2, 0)
            proc(u0 + 1, 1, p)
            return carry

        lax.fori_loop(0, nu // 2, body, 0)
        for s in range(2):
            drain_write(s)

    return k


def kernel(item_inputs, table):
    b, h = item_inputs.shape
    v, d = table.shape
    idx1 = item_inputs.T.reshape(b * h).astype(jnp.int32)
    out5 = _make_lookup(b, h)(table, idx1)
    return out5.transpose(2, 4, 0, 1, 3).reshape(b, h, d)
